# baseline (speedup 1.0000x reference)
import functools

import jax
import jax.numpy as jnp
from jax import lax
from jax.experimental import pallas as pl
from jax.experimental.pallas import tpu as pltpu
from jax.experimental.pallas import tpu_sc as plsc

N_NODES = 10000
D_IN = 128
D_OUT = 128
N_EDGES = 320000

NC = 2
NS = 16
LANES = 16

EPT = N_EDGES // NS
BLK = 125
NBLK = EPT // BLK
SB = 4
NSUP = NBLK // SB
RPT = N_NODES // NS


def _sc_segment_sum(feat, idx2, zrows, zdeg, ones_blk):
    mesh = plsc.VectorSubcoreMesh(core_axis_name="c", subcore_axis_name="s")

    @functools.partial(
        pl.kernel,
        out_type=(
            jax.ShapeDtypeStruct((N_NODES, D_IN), jnp.float32),
            jax.ShapeDtypeStruct((N_NODES, D_IN), jnp.float32),
            jax.ShapeDtypeStruct((N_NODES, LANES), jnp.float32),
            jax.ShapeDtypeStruct((N_NODES, LANES), jnp.float32),
        ),
        mesh=mesh,
        scratch_types=[
            pltpu.VMEM((2, SB, 2, BLK), jnp.int32),
            pltpu.VMEM((2, BLK, D_IN), jnp.float32),
            pltpu.VMEM((BLK, LANES), jnp.float32),
            pltpu.VMEM_SHARED((N_NODES, D_IN), jnp.float32),
            pltpu.VMEM_SHARED((N_NODES, LANES), jnp.float32),
            pltpu.SemaphoreType.DMA,
            pltpu.SemaphoreType.DMA,
            pltpu.SemaphoreType.DMA,
            pltpu.SemaphoreType.DMA,
            pltpu.SemaphoreType.DMA,
            pltpu.SemaphoreType.DMA,
            pltpu.SemaphoreType.DMA,
            pltpu.SemaphoreType.DMA,
        ],
        compiler_params=pltpu.CompilerParams(use_tc_tiling_on_sc=False),
    )
    def seg_sum(feat_hbm, idx_hbm, zrows_hbm, zdeg_hbm, ones_hbm,
                sf_out, sl_out, df_out, dl_out,
                idxb, rows, onesb, acc, dacc,
                si0, si1, sg0, sg1, ss0, ss1, sd0, sd1):
        r = lax.axis_index("c")
        s = lax.axis_index("s")
        rowbase = (r * NS + s) * NBLK
        isems = (si0, si1)
        gsems = (sg0, sg1)
        ssems = (ss0, ss1)
        dsems = (sd0, sd1)

        pltpu.sync_copy(zrows_hbm, acc.at[pl.ds(s * RPT, RPT)])
        pltpu.sync_copy(zdeg_hbm, dacc.at[pl.ds(s * RPT, RPT)])
        pltpu.sync_copy(ones_hbm, onesb)

        def isup_issue(m, p):
            pltpu.async_copy(idx_hbm.at[pl.ds(rowbase + m * SB, SB)],
                             idxb.at[p], isems[p])

        def isup_wait(p):
            pltpu.make_async_copy(idx_hbm.at[pl.ds(rowbase, SB)], idxb.at[p],
                                  isems[p]).wait()

        def gath_issue(p, q, b):
            pltpu.async_copy(feat_hbm.at[idxb.at[p, q, 0]], rows.at[b],
                             gsems[b])

        def gath_wait(b):
            pltpu.make_async_copy(feat_hbm.at[idxb.at[0, 0, 0]], rows.at[b],
                                  gsems[b]).wait()

        def scat_issue(p, q, b):
            pltpu.async_copy(rows.at[b], acc.at[idxb.at[p, q, 1]], ssems[b],
                             add=True)
            pltpu.async_copy(onesb, dacc.at[idxb.at[p, q, 1]], dsems[b],
                             add=True)

        def scat_wait(b):
            pltpu.make_async_copy(rows.at[b], acc.at[idxb.at[0, 0, 1]],
                                  ssems[b]).wait()
            pltpu.make_async_copy(onesb, dacc.at[idxb.at[0, 0, 1]],
                                  dsems[b]).wait()

        def super_body(m, S):
            T = 1 - S
            gath_wait(0)
            scat_issue(S, 0, 0)

            @pl.when(m * SB > 0)
            def _():
                scat_wait(1)

            @pl.when((m >= 1) & (m + 1 < NSUP))
            def _():
                isup_issue(m + 1, T)

            gath_issue(S, 1, 1)
            gath_wait(1)
            scat_issue(S, 1, 1)
            scat_wait(0)
            gath_issue(S, 2, 0)
            gath_wait(0)
            scat_issue(S, 2, 0)
            scat_wait(1)
            gath_issue(S, 3, 1)
            gath_wait(1)
            scat_issue(S, 3, 1)

            @pl.when(m + 1 < NSUP)
            def _():
                isup_wait(T)

            scat_wait(0)

            @pl.when(m + 1 < NSUP)
            def _():
                gath_issue(T, 0, 0)

        isup_issue(0, 0)
        isup_issue(1, 1)
        plsc.subcore_barrier()
        isup_wait(0)
        gath_issue(0, 0, 0)

        @pl.loop(0, NSUP - 2, step=2)
        def _(g):
            super_body(g, 0)
            super_body(g + 1, 1)

        super_body(NSUP - 2, 0)
        super_body(NSUP - 1, 1)
        scat_wait(1)

        plsc.subcore_barrier()

        rs = pl.ds(s * RPT, RPT)

        @pl.when(r == 0)
        def _():
            pltpu.sync_copy(acc.at[rs], sf_out.at[rs])
            pltpu.sync_copy(dacc.at[rs], df_out.at[rs])

        @pl.when(r == 1)
        def _():
            pltpu.sync_copy(acc.at[rs], sl_out.at[rs])
            pltpu.sync_copy(dacc.at[rs], dl_out.at[rs])

    return seg_sum(feat, idx2, zrows, zdeg, ones_blk)


def _tc_combine_body(feat_ref, sf_ref, sl_ref, df_ref, dl_ref,
                     w0_ref, b0_ref, wf_ref, bf_ref, wl_ref, bl_ref,
                     out_ref):
    df = df_ref[:, 0:1]
    dl = dl_ref[:, 0:1]
    dims = (((1,), (1,)), ((), ()))
    hp = jax.lax.Precision.HIGHEST
    h = lax.dot_general(feat_ref[...], w0_ref[...], dims, precision=hp,
                        preferred_element_type=jnp.float32) + b0_ref[...][None, :]
    hf = lax.dot_general(sf_ref[...], wf_ref[...], dims, precision=hp,
                         preferred_element_type=jnp.float32) \
        + df * bf_ref[...][None, :]
    h = h + hf / jnp.maximum(df, 1.0)
    hl = lax.dot_general(sl_ref[...], wl_ref[...], dims, precision=hp,
                         preferred_element_type=jnp.float32) \
        + dl * bl_ref[...][None, :]
    h = h + hl / jnp.maximum(dl, 1.0)
    out_ref[...] = h


def _tc_combine(feat, s_f, s_l, deg_f, deg_l, W0, b0, Wf, bf, Wl, bl):
    blk = 5000
    grid = N_NODES // blk
    wspec = pl.BlockSpec((D_IN, D_OUT), lambda i: (0, 0))
    bspec = pl.BlockSpec((D_OUT,), lambda i: (0,))
    return pl.pallas_call(
        _tc_combine_body,
        grid=(grid,),
        in_specs=[
            pl.BlockSpec((blk, D_IN), lambda i: (i, 0)),
            pl.BlockSpec((blk, D_IN), lambda i: (i, 0)),
            pl.BlockSpec((blk, D_IN), lambda i: (i, 0)),
            pl.BlockSpec((blk, LANES), lambda i: (i, 0)),
            pl.BlockSpec((blk, LANES), lambda i: (i, 0)),
            wspec, bspec, wspec, bspec, wspec, bspec,
        ],
        out_specs=pl.BlockSpec((blk, D_OUT), lambda i: (i, 0)),
        out_shape=jax.ShapeDtypeStruct((N_NODES, D_OUT), jnp.float32),
    )(feat, s_f, s_l, deg_f, deg_l, W0, b0, Wf, bf, Wl, bl)


def kernel(feat, edge_index_follows, edge_index_likes,
           W0, b0, W_follows, b_follows, W_likes, b_likes):
    idx2 = jnp.concatenate([edge_index_follows, edge_index_likes], axis=1)
    idx2 = idx2.reshape(2, NC * N_EDGES // BLK, BLK).transpose(1, 0, 2)
    zrows = jnp.zeros((RPT, D_IN), dtype=jnp.float32)
    zdeg = jnp.zeros((RPT, LANES), dtype=jnp.float32)
    ones_blk = jnp.ones((BLK, LANES), dtype=jnp.float32)

    s_f, s_l, deg_f, deg_l = _sc_segment_sum(feat, idx2, zrows, zdeg, ones_blk)
    return _tc_combine(feat, s_f, s_l, deg_f, deg_l,
                       W0, b0, W_follows, b_follows, W_likes, b_likes)


# speedup vs baseline: 1.1045x; 1.1045x over previous
import functools

import jax
import jax.numpy as jnp
from jax import lax
from jax.experimental import pallas as pl
from jax.experimental.pallas import tpu as pltpu
from jax.experimental.pallas import tpu_sc as plsc

N_NODES = 10000
D_IN = 128
D_OUT = 128
N_EDGES = 320000

NC = 2
NS = 16
LANES = 16

EPT = N_EDGES // NS
BLK = 125
NBLK = EPT // BLK
SB = 4
NSUP = NBLK // SB
RPT = N_NODES // NS


def _sc_segment_sum(feat, idx2, zrows, zdeg, ones_blk):
    mesh = plsc.VectorSubcoreMesh(core_axis_name="c", subcore_axis_name="s")

    @functools.partial(
        pl.kernel,
        out_type=(
            jax.ShapeDtypeStruct((N_NODES, D_IN), jnp.float32),
            jax.ShapeDtypeStruct((N_NODES, D_IN), jnp.float32),
            jax.ShapeDtypeStruct((N_NODES, LANES), jnp.float32),
            jax.ShapeDtypeStruct((N_NODES, LANES), jnp.float32),
        ),
        mesh=mesh,
        scratch_types=[
            pltpu.VMEM((2, 2, SB, BLK), jnp.int32),
            pltpu.VMEM((2, BLK, D_IN), jnp.float32),
            pltpu.VMEM((BLK, LANES), jnp.float32),
            pltpu.VMEM_SHARED((N_NODES, D_IN), jnp.float32),
            pltpu.VMEM_SHARED((N_NODES, LANES), jnp.float32),
            pltpu.SemaphoreType.DMA,
            pltpu.SemaphoreType.DMA,
            pltpu.SemaphoreType.DMA,
            pltpu.SemaphoreType.DMA,
            pltpu.SemaphoreType.DMA,
            pltpu.SemaphoreType.DMA,
            pltpu.SemaphoreType.DMA,
            pltpu.SemaphoreType.DMA,
            pltpu.SemaphoreType.DMA,
            pltpu.SemaphoreType.DMA,
        ],
        compiler_params=pltpu.CompilerParams(use_tc_tiling_on_sc=False),
    )
    def seg_sum(feat_hbm, idx_hbm, zrows_hbm, zdeg_hbm, ones_hbm,
                sf_out, sl_out, df_out, dl_out,
                idxb, rows, onesb, acc, dacc,
                si0, si1, sj0, sj1, sg0, sg1, ss0, ss1, sd0, sd1):
        r = lax.axis_index("c")
        s = lax.axis_index("s")
        rowbase = (r * NS + s) * NBLK
        isems = (si0, si1)
        jsems = (sj0, sj1)
        gsems = (sg0, sg1)
        ssems = (ss0, ss1)
        dsems = (sd0, sd1)

        pltpu.sync_copy(zrows_hbm, acc.at[pl.ds(s * RPT, RPT)])
        pltpu.sync_copy(zdeg_hbm, dacc.at[pl.ds(s * RPT, RPT)])
        pltpu.sync_copy(ones_hbm, onesb)

        def isup_issue(m, p):
            pltpu.async_copy(idx_hbm.at[0, pl.ds(rowbase + m * SB, SB)],
                             idxb.at[p, 0], isems[p])
            pltpu.async_copy(idx_hbm.at[1, pl.ds(rowbase + m * SB, SB)],
                             idxb.at[p, 1], jsems[p])

        def isup_wait(p):
            pltpu.make_async_copy(idx_hbm.at[0, pl.ds(rowbase, SB)],
                                  idxb.at[p, 0], isems[p]).wait()
            pltpu.make_async_copy(idx_hbm.at[1, pl.ds(rowbase, SB)],
                                  idxb.at[p, 1], jsems[p]).wait()

        def gath_issue(p, q, b):
            pltpu.async_copy(feat_hbm.at[idxb.at[p, 0, q]], rows.at[b],
                             gsems[b])

        def gath_wait(b):
            pltpu.make_async_copy(feat_hbm.at[idxb.at[0, 0, 0]], rows.at[b],
                                  gsems[b]).wait()

        def scat_issue(p, q, b):
            pltpu.async_copy(rows.at[b], acc.at[idxb.at[p, 1, q]], ssems[b],
                             add=True)
            pltpu.async_copy(onesb, dacc.at[idxb.at[p, 1, q]], dsems[b],
                             add=True)

        def scat_wait(b):
            pltpu.make_async_copy(rows.at[b], acc.at[idxb.at[0, 1, 0]],
                                  ssems[b]).wait()
            pltpu.make_async_copy(onesb, dacc.at[idxb.at[0, 1, 0]],
                                  dsems[b]).wait()

        def super_body(m, S):
            T = 1 - S
            gath_wait(0)
            scat_issue(S, 0, 0)

            @pl.when(m * SB > 0)
            def _():
                scat_wait(1)

            @pl.when((m >= 1) & (m + 1 < NSUP))
            def _():
                isup_issue(m + 1, T)

            gath_issue(S, 1, 1)
            gath_wait(1)
            scat_issue(S, 1, 1)
            scat_wait(0)
            gath_issue(S, 2, 0)
            gath_wait(0)
            scat_issue(S, 2, 0)
            scat_wait(1)
            gath_issue(S, 3, 1)
            gath_wait(1)
            scat_issue(S, 3, 1)

            @pl.when(m + 1 < NSUP)
            def _():
                isup_wait(T)

            scat_wait(0)

            @pl.when(m + 1 < NSUP)
            def _():
                gath_issue(T, 0, 0)

        isup_issue(0, 0)
        isup_issue(1, 1)
        plsc.subcore_barrier()
        isup_wait(0)
        gath_issue(0, 0, 0)

        @pl.loop(0, NSUP - 2, step=2)
        def _(g):
            super_body(g, 0)
            super_body(g + 1, 1)

        super_body(NSUP - 2, 0)
        super_body(NSUP - 1, 1)
        scat_wait(1)

        plsc.subcore_barrier()

        rs = pl.ds(s * RPT, RPT)

        @pl.when(r == 0)
        def _():
            pltpu.sync_copy(acc.at[rs], sf_out.at[rs])
            pltpu.sync_copy(dacc.at[rs], df_out.at[rs])

        @pl.when(r == 1)
        def _():
            pltpu.sync_copy(acc.at[rs], sl_out.at[rs])
            pltpu.sync_copy(dacc.at[rs], dl_out.at[rs])

    return seg_sum(feat, idx2, zrows, zdeg, ones_blk)


def _tc_combine_body(feat_ref, sf_ref, sl_ref, df_ref, dl_ref,
                     w0_ref, b0_ref, wf_ref, bf_ref, wl_ref, bl_ref,
                     out_ref):
    df = df_ref[:, 0:1]
    dl = dl_ref[:, 0:1]
    dims = (((1,), (1,)), ((), ()))
    hp = jax.lax.Precision.HIGHEST
    h = lax.dot_general(feat_ref[...], w0_ref[...], dims, precision=hp,
                        preferred_element_type=jnp.float32) + b0_ref[...][None, :]
    hf = lax.dot_general(sf_ref[...], wf_ref[...], dims, precision=hp,
                         preferred_element_type=jnp.float32) \
        + df * bf_ref[...][None, :]
    h = h + hf / jnp.maximum(df, 1.0)
    hl = lax.dot_general(sl_ref[...], wl_ref[...], dims, precision=hp,
                         preferred_element_type=jnp.float32) \
        + dl * bl_ref[...][None, :]
    h = h + hl / jnp.maximum(dl, 1.0)
    out_ref[...] = h


def _tc_combine(feat, s_f, s_l, deg_f, deg_l, W0, b0, Wf, bf, Wl, bl):
    blk = 2000
    grid = N_NODES // blk
    wspec = pl.BlockSpec((D_IN, D_OUT), lambda i: (0, 0))
    bspec = pl.BlockSpec((D_OUT,), lambda i: (0,))
    return pl.pallas_call(
        _tc_combine_body,
        grid=(grid,),
        in_specs=[
            pl.BlockSpec((blk, D_IN), lambda i: (i, 0)),
            pl.BlockSpec((blk, D_IN), lambda i: (i, 0)),
            pl.BlockSpec((blk, D_IN), lambda i: (i, 0)),
            pl.BlockSpec((blk, LANES), lambda i: (i, 0)),
            pl.BlockSpec((blk, LANES), lambda i: (i, 0)),
            wspec, bspec, wspec, bspec, wspec, bspec,
        ],
        out_specs=pl.BlockSpec((blk, D_OUT), lambda i: (i, 0)),
        out_shape=jax.ShapeDtypeStruct((N_NODES, D_OUT), jnp.float32),
    )(feat, s_f, s_l, deg_f, deg_l, W0, b0, Wf, bf, Wl, bl)


def kernel(feat, edge_index_follows, edge_index_likes,
           W0, b0, W_follows, b_follows, W_likes, b_likes):
    idx2 = jnp.concatenate([edge_index_follows, edge_index_likes], axis=1)
    idx2 = idx2.reshape(2, NC * N_EDGES // BLK, BLK)
    zrows = jnp.zeros((RPT, D_IN), dtype=jnp.float32)
    zdeg = jnp.zeros((RPT, LANES), dtype=jnp.float32)
    ones_blk = jnp.ones((BLK, LANES), dtype=jnp.float32)

    s_f, s_l, deg_f, deg_l = _sc_segment_sum(feat, idx2, zrows, zdeg, ones_blk)
    return _tc_combine(feat, s_f, s_l, deg_f, deg_l,
                       W0, b0, W_follows, b_follows, W_likes, b_likes)
